# Initial kernel scaffold; baseline (speedup 1.0000x reference)
import jax
import jax.numpy as jnp
from jax.experimental import pallas as pl


def kernel(tgt, pos_index, tgt_token_embedding):
    raise NotImplementedError("write your pallas kernel here")


# speedup vs baseline: 1.2380x; 1.2380x over previous
import functools

import jax
import jax.numpy as jnp
import numpy as np
from jax import lax
from jax.experimental import pallas as pl
from jax.experimental.pallas import tpu as pltpu
from jax.experimental.pallas import tpu_sc as plsc


def _div_term_np(dim, lambda_max=10000.0, lambda_min=1e-05):
    base = lambda_max / (2 * np.pi)
    scale = lambda_min / lambda_max
    return (base * scale ** (np.arange(0, dim, 2) / dim)).astype(np.float32)


def _sc_gather(table, idx, B, D, chunk=16):
    info = plsc.get_sparse_core_info()
    NC, NS = info.num_cores, info.num_subcores
    NW = NC * NS
    assert B % NW == 0
    b_per_w = B // NW
    assert b_per_w % chunk == 0
    n_chunks = b_per_w // chunk

    mesh = plsc.VectorSubcoreMesh(core_axis_name="c", subcore_axis_name="s")

    @functools.partial(
        pl.kernel,
        out_type=jax.ShapeDtypeStruct((B, D), jnp.float32),
        mesh=mesh,
        scratch_types=[
            pltpu.VMEM((b_per_w,), jnp.int32),
            pltpu.VMEM((chunk, D), jnp.float32),
            pltpu.VMEM((chunk, D), jnp.float32),
            pltpu.SemaphoreType.DMA,
            pltpu.SemaphoreType.DMA,
            pltpu.SemaphoreType.DMA,
            pltpu.SemaphoreType.DMA,
        ],
    )
    def gather_kernel(table_hbm, idx_hbm, out_hbm, idx_v, buf0, buf1,
                      gsem0, gsem1, ssem0, ssem1):
        wid = lax.axis_index("s") * NC + lax.axis_index("c")
        base = wid * b_per_w
        pltpu.sync_copy(idx_hbm.at[pl.ds(base, b_per_w)], idx_v)

        bufs = (buf0, buf1)
        gsems = (gsem0, gsem1)
        ssems = (ssem0, ssem1)

        def start_gather(c):
            b = c % 2
            return pltpu.async_copy(
                table_hbm.at[idx_v.at[pl.ds(c * chunk, chunk)]],
                bufs[b], gsems[b])

        def start_scatter(c):
            b = c % 2
            return pltpu.async_copy(
                bufs[b], out_hbm.at[pl.ds(base + c * chunk, chunk)], ssems[b])

        gather_h = [None] * n_chunks
        scatter_h = [None] * n_chunks
        gather_h[0] = start_gather(0)
        for c in range(n_chunks):
            if c + 1 < n_chunks:
                if c - 1 >= 0:
                    scatter_h[c - 1].wait()
                gather_h[c + 1] = start_gather(c + 1)
            gather_h[c].wait()
            scatter_h[c] = start_scatter(c)
        if n_chunks >= 2:
            scatter_h[n_chunks - 2].wait()
        scatter_h[n_chunks - 1].wait()

    return gather_kernel(table, idx)


def _pe_add_body(g_ref, pos_ref, rec_ref, out_ref):
    x = pos_ref[...] * rec_ref[...]
    g = g_ref[...]
    h = rec_ref.shape[1]
    out_ref[:, :h] = g[:, :h] + jnp.sin(x)
    out_ref[:, h:] = g[:, h:] + jnp.cos(x)


def _tc_pe_add(g, pos, div, B, D, R=256):
    grid = (B // R,)
    return pl.pallas_call(
        _pe_add_body,
        grid=grid,
        in_specs=[
            pl.BlockSpec((R, D), lambda i: (i, 0)),
            pl.BlockSpec((R, 1), lambda i: (i, 0)),
            pl.BlockSpec((1, D // 2), lambda i: (0, 0)),
        ],
        out_specs=pl.BlockSpec((R, D), lambda i: (i, 0)),
        out_shape=jax.ShapeDtypeStruct((B, D), jnp.float32),
        input_output_aliases={0: 0},
    )(g, pos, div)


def kernel(tgt, pos_index, tgt_token_embedding):
    Bt, S = tgt.shape
    V, D = tgt_token_embedding.shape
    B = Bt * S
    idx = tgt.reshape(B).astype(jnp.int32)
    g = _sc_gather(tgt_token_embedding, idx, B, D)
    pos = pos_index.reshape(B, 1)
    rec = jnp.asarray(np.float32(1.0) / _div_term_np(D)).reshape(1, D // 2)
    out = _tc_pe_add(g, pos, rec, B, D)
    return out.reshape(Bt, S, D)


# speedup vs baseline: 1.3415x; 1.0836x over previous
import functools

import jax
import jax.numpy as jnp
import numpy as np
from jax import lax
from jax.experimental import pallas as pl
from jax.experimental.pallas import tpu as pltpu
from jax.experimental.pallas import tpu_sc as plsc


def _div_term_np(dim, lambda_max=10000.0, lambda_min=1e-05):
    base = lambda_max / (2 * np.pi)
    scale = lambda_min / lambda_max
    return (base * scale ** (np.arange(0, dim, 2) / dim)).astype(np.float32)


def _sc_gather(table, idx, B, D, chunk=16):
    info = plsc.get_sparse_core_info()
    NC, NS = info.num_cores, info.num_subcores
    NW = NC * NS
    assert B % NW == 0
    b_per_w = B // NW
    assert b_per_w % chunk == 0
    n_chunks = b_per_w // chunk

    mesh = plsc.VectorSubcoreMesh(core_axis_name="c", subcore_axis_name="s")

    @functools.partial(
        pl.kernel,
        out_type=jax.ShapeDtypeStruct((B, D), jnp.float32),
        mesh=mesh,
        scratch_types=[
            pltpu.VMEM((b_per_w,), jnp.int32),
            pltpu.VMEM((chunk, D), jnp.float32),
            pltpu.VMEM((chunk, D), jnp.float32),
            pltpu.SemaphoreType.DMA,
            pltpu.SemaphoreType.DMA,
            pltpu.SemaphoreType.DMA,
            pltpu.SemaphoreType.DMA,
        ],
    )
    def gather_kernel(table_hbm, idx_hbm, out_hbm, idx_v, buf0, buf1,
                      gsem0, gsem1, ssem0, ssem1):
        wid = lax.axis_index("s") * NC + lax.axis_index("c")
        base = wid * b_per_w
        pltpu.sync_copy(idx_hbm.at[pl.ds(base, b_per_w)], idx_v)

        bufs = (buf0, buf1)
        gsems = (gsem0, gsem1)
        ssems = (ssem0, ssem1)

        def start_gather(c):
            b = c % 2
            return pltpu.async_copy(
                table_hbm.at[idx_v.at[pl.ds(c * chunk, chunk)]],
                bufs[b], gsems[b])

        def start_scatter(c):
            b = c % 2
            return pltpu.async_copy(
                bufs[b], out_hbm.at[pl.ds(base + c * chunk, chunk)], ssems[b])

        gather_h = [None] * n_chunks
        scatter_h = [None] * n_chunks
        gather_h[0] = start_gather(0)
        for c in range(n_chunks):
            if c + 1 < n_chunks:
                if c - 1 >= 0:
                    scatter_h[c - 1].wait()
                gather_h[c + 1] = start_gather(c + 1)
            gather_h[c].wait()
            scatter_h[c] = start_scatter(c)
        if n_chunks >= 2:
            scatter_h[n_chunks - 2].wait()
        scatter_h[n_chunks - 1].wait()

    return gather_kernel(table, idx)


def _pe_add_chunk_body(g_ref, pos_ref, rec_ref, out_ref):
    x = pos_ref[...] * rec_ref[...]
    g = g_ref[...]
    h = rec_ref.shape[1]
    out_ref[:, :h] = g[:, :h] + jnp.sin(x)
    out_ref[:, h:] = g[:, h:] + jnp.cos(x)


def _pe_add_chunk_body_alias(prev_ref, g_ref, pos_ref, rec_ref, out_ref):
    del prev_ref
    _pe_add_chunk_body(g_ref, pos_ref, rec_ref, out_ref)


def _tc_pe_add_chunk(prev, g, pos, rec, B, D, row0, R=512):
    chunk = g.shape[0]
    grid = (chunk // R,)
    g_spec = pl.BlockSpec((R, D), lambda i: (i, 0))
    pos_spec = pl.BlockSpec((R, 1), lambda i: (i, 0))
    rec_spec = pl.BlockSpec((1, D // 2), lambda i: (0, 0))
    out_spec = pl.BlockSpec((R, D), lambda i, _r0=row0 // R: (i + _r0, 0))
    out_shape = jax.ShapeDtypeStruct((B, D), jnp.float32)
    if prev is None:
        return pl.pallas_call(
            _pe_add_chunk_body, grid=grid,
            in_specs=[g_spec, pos_spec, rec_spec],
            out_specs=out_spec, out_shape=out_shape,
        )(g, pos, rec)
    return pl.pallas_call(
        _pe_add_chunk_body_alias, grid=grid,
        in_specs=[pl.BlockSpec(memory_space=pl.ANY), g_spec, pos_spec, rec_spec],
        out_specs=out_spec, out_shape=out_shape,
        input_output_aliases={0: 0},
    )(prev, g, pos, rec)


def _pe_add_body(g_ref, pos_ref, rec_ref, out_ref):
    x = pos_ref[...] * rec_ref[...]
    g = g_ref[...]
    h = rec_ref.shape[1]
    out_ref[:, :h] = g[:, :h] + jnp.sin(x)
    out_ref[:, h:] = g[:, h:] + jnp.cos(x)


def _tc_pe_add(g, pos, div, B, D, R=256):
    grid = (B // R,)
    return pl.pallas_call(
        _pe_add_body,
        grid=grid,
        in_specs=[
            pl.BlockSpec((R, D), lambda i: (i, 0)),
            pl.BlockSpec((R, 1), lambda i: (i, 0)),
            pl.BlockSpec((1, D // 2), lambda i: (0, 0)),
        ],
        out_specs=pl.BlockSpec((R, D), lambda i: (i, 0)),
        out_shape=jax.ShapeDtypeStruct((B, D), jnp.float32),
        input_output_aliases={0: 0},
    )(g, pos, div)


def kernel(tgt, pos_index, tgt_token_embedding):
    Bt, S = tgt.shape
    V, D = tgt_token_embedding.shape
    B = Bt * S
    idx = tgt.reshape(B).astype(jnp.int32)
    pos = pos_index.reshape(B, 1)
    rec = jnp.asarray(np.float32(1.0) / _div_term_np(D)).reshape(1, D // 2)

    K = 4
    C = B // K
    gs = [_sc_gather(tgt_token_embedding, lax.slice(idx, (k * C,), ((k + 1) * C,)),
                     C, D) for k in range(K)]
    out = None
    for k in range(K):
        out = _tc_pe_add_chunk(out, gs[k],
                               lax.slice(pos, (k * C, 0), ((k + 1) * C, 1)),
                               rec, B, D, k * C)
    return out.reshape(Bt, S, D)
